# Initial kernel scaffold; baseline (speedup 1.0000x reference)
import jax
import jax.numpy as jnp
from jax.experimental import pallas as pl


def kernel(x, table, W, b):
    raise NotImplementedError("write your pallas kernel here")


# speedup vs baseline: 34.0100x; 34.0100x over previous
import functools

import jax
import jax.numpy as jnp
from jax import lax
from jax.experimental import pallas as pl
from jax.experimental.pallas import tpu as pltpu
from jax.experimental.pallas import tpu_sc as plsc

B = 16384
L = 50
EMBED = 16
NCLS = 3

NC = 2
NS = 16
NW = NC * NS
S_PER_W = B // NW
BURST = 100
CHUNK_S = 32
CHUNK_ROWS = CHUNK_S * L
BURSTS = CHUNK_ROWS // BURST
N_CHUNKS = S_PER_W // CHUNK_S
SG = 8
X_ROWS_PER_W = S_PER_W * L // BURST


def _body(x2d, table, w3, b16, out,
          idx0, idx1, rows0, rows1, wv, bv, stage, outbuf, sem0, sem1):
  wid = lax.axis_index("s") * NC + lax.axis_index("c")
  xr0 = wid * X_ROWS_PER_W

  pltpu.sync_copy(w3, wv)
  pltpu.sync_copy(b16, bv)

  idx_bufs = (idx0, idx1)
  rows_bufs = (rows0, rows1)
  sems = (sem0, sem1)

  def fire(g, slot):
    pltpu.sync_copy(x2d.at[pl.ds(xr0 + g * BURSTS, BURSTS)], idx_bufs[slot])
    for j in range(BURSTS):
      pltpu.async_copy(table.at[idx_bufs[slot].at[j]],
                       rows_bufs[slot].at[pl.ds(j * BURST, BURST)],
                       sems[slot])

  def drain(slot):
    pltpu.make_async_copy(table.at[pl.ds(0, CHUNK_ROWS)],
                          rows_bufs[slot], sems[slot]).wait()

  lane = lax.iota(jnp.int32, EMBED)

  def _perm(v, idx):
    return lax.gather(
        v, idx[:, None],
        lax.GatherDimensionNumbers(
            offset_dims=(), collapsed_slice_dims=(0,), start_index_map=(0,)),
        (1,), mode=lax.GatherScatterMode.PROMISE_IN_BOUNDS)

  def _pairsum(u, v, s):
    m = (lane & s) == 0
    return jnp.where(m, u + _perm(u, lane ^ s), v + _perm(v, lane ^ s))

  def _reduce8(vs):
    while len(vs) > 1:
      s = {8: 1, 4: 2, 2: 4}[len(vs)]
      vs = [_pairsum(vs[2 * i], vs[2 * i + 1], s) for i in range(len(vs) // 2)]
    f = vs[0]
    return f + _perm(f, lane ^ SG)

  def compute(g, slot):
    rows = rows_bufs[slot]

    def sg_accs(m):
      def l_body(l, accs):
        w0 = wv[0, l]
        w1 = wv[1, l]
        w2 = wv[2, l]
        base = m * (SG * L) + l
        new = []
        for si in range(SG):
          row = rows[base + si * L]
          a0, a1, a2 = accs[3 * si:3 * si + 3]
          new.extend((a0 + row * w0, a1 + row * w1, a2 + row * w2))
        return tuple(new)

      zeros = jnp.zeros((EMBED,), jnp.float32)
      accs = lax.fori_loop(0, L, l_body, (zeros,) * (3 * SG))
      return [_reduce8([accs[3 * si + c] for si in range(SG)])
              for c in range(NCLS)]

    def pair_body(mm, _):
      fa = sg_accs(2 * mm)
      fb = sg_accs(2 * mm + 1)
      low = lane < SG
      for c in range(NCLS):
        stage[c, pl.ds(g * CHUNK_S + mm * EMBED, EMBED)] = (
            jnp.where(low, fa[c], fb[c]))
      return 0

    lax.fori_loop(0, CHUNK_S // (2 * SG), pair_body, 0)

  fire(0, 0)

  def gg_body(gg, _):
    g0 = 2 * gg
    fire(g0 + 1, 1)
    drain(0)
    compute(g0, 0)

    @pl.when(g0 + 2 < N_CHUNKS)
    def _():
      fire(g0 + 2, 0)

    drain(1)
    compute(g0 + 1, 1)
    return 0

  lax.fori_loop(0, N_CHUNKS // 2, gg_body, 0)

  def sm_body(r, _):
    sl = pl.ds(r * EMBED, EMBED)
    a = stage[0, sl] + bv[0]
    bb = stage[1, sl] + bv[1]
    cc = stage[2, sl] + bv[2]
    m = jnp.maximum(jnp.maximum(a, bb), cc)
    s = jnp.exp(a - m) + jnp.exp(bb - m) + jnp.exp(cc - m)
    t = (s - 1.0) / (s + 1.0)
    t2 = t * t
    y = 2.0 * t * (1.0 + t2 * (0.33333334 + t2 * 0.2))
    y = y + s * jnp.exp(-y) - 1.0
    y = y + s * jnp.exp(-y) - 1.0
    outbuf[0, sl] = a - m - y
    outbuf[1, sl] = bb - m - y
    outbuf[2, sl] = cc - m - y
    return 0

  lax.fori_loop(0, S_PER_W // EMBED, sm_body, 0)

  for c in range(NCLS):
    pltpu.sync_copy(outbuf.at[pl.ds(c, 1)],
                    out.at[pl.ds(c, 1), pl.ds(wid * S_PER_W, S_PER_W)])


@functools.partial(jax.jit, donate_argnums=())
def kernel(x, table, W, b):
  x2d = x.reshape(B * L // BURST, BURST).astype(jnp.int32)
  w3 = W.reshape(NCLS, L, EMBED)
  b16 = jnp.broadcast_to(b[:, None], (NCLS, EMBED))

  mesh = plsc.VectorSubcoreMesh(core_axis_name="c", subcore_axis_name="s")
  run = pl.kernel(
      _body,
      out_type=jax.ShapeDtypeStruct((NCLS, B), jnp.float32),
      mesh=mesh,
      compiler_params=pltpu.CompilerParams(use_tc_tiling_on_sc=False),
      scratch_types=[
          pltpu.VMEM((BURSTS, BURST), jnp.int32),
          pltpu.VMEM((BURSTS, BURST), jnp.int32),
          pltpu.VMEM((CHUNK_ROWS, EMBED), jnp.float32),
          pltpu.VMEM((CHUNK_ROWS, EMBED), jnp.float32),
          pltpu.VMEM((NCLS, L, EMBED), jnp.float32),
          pltpu.VMEM((NCLS, EMBED), jnp.float32),
          pltpu.VMEM((NCLS, S_PER_W), jnp.float32),
          pltpu.VMEM((NCLS, S_PER_W), jnp.float32),
          pltpu.SemaphoreType.DMA,
          pltpu.SemaphoreType.DMA,
      ],
  )
  res = run(x2d, table, w3, b16)
  return res.T
